# baseline (speedup 1.0000x reference)
import math

import jax
import jax.numpy as jnp
from jax.experimental import pallas as pl
from jax.experimental.pallas import tpu as pltpu

_SIZE = 100000
_SMOOTHING = 0.1
_CONF = 1.0 - _SMOOTHING
_N = 1024
_FILL = _SMOOTHING / (_SIZE - 1)
_CONST = _N * ((_SIZE - 1) * _FILL * math.log(_FILL) + _CONF * math.log(_CONF))

_BR = 8
_NBUF = 8
_NCHUNK = _N // _BR


def _body(tgt_ref, x_hbm, o_ref, buf, sems, acc):
    def start(c, slot):
        pltpu.make_async_copy(
            x_hbm.at[pl.ds(c * _BR, _BR), :], buf.at[slot], sems.at[slot]
        ).start()

    for b in range(_NBUF):
        start(b, b)
    acc[0] = jnp.float32(0.0)

    def outer(g, _):
        for b in range(_NBUF):
            c = g * _NBUF + b
            pltpu.make_async_copy(
                x_hbm.at[pl.ds(c * _BR, _BR), :], buf.at[b], sems.at[b]
            ).wait()
            x = buf[b]
            t = tgt_ref[pl.ds(c, 1), 0, :]
            cols = jax.lax.broadcasted_iota(jnp.int32, x.shape, 1)
            w = jnp.where(cols == t.reshape(_BR, 1), jnp.float32(_CONF),
                          jnp.float32(_FILL))
            acc[0] += jnp.sum(w * x)

            @pl.when(c + _NBUF < _NCHUNK)
            def _refill():
                start(c + _NBUF, b)
        return 0

    jax.lax.fori_loop(0, _NCHUNK // _NBUF, outer, 0)
    o_ref[0, 0] = jnp.float32(_CONST) - acc[0]


def kernel(x, target):
    tgt3 = target.astype(jnp.int32).reshape(_NCHUNK, 1, _BR)
    out = pl.pallas_call(
        _body,
        in_specs=[
            pl.BlockSpec(memory_space=pltpu.VMEM),
            pl.BlockSpec(memory_space=pl.ANY),
        ],
        out_specs=pl.BlockSpec(memory_space=pltpu.SMEM),
        out_shape=jax.ShapeDtypeStruct((1, 1), jnp.float32),
        scratch_shapes=[
            pltpu.VMEM((_NBUF, _BR, _SIZE), jnp.float32),
            pltpu.SemaphoreType.DMA((_NBUF,)),
            pltpu.SMEM((1,), jnp.float32),
        ],
    )(tgt3, x)
    return out[0, 0]
